# baseline (speedup 1.0000x reference)
import jax
import jax.numpy as jnp
from jax import lax
from jax.experimental import pallas as pl
from jax.experimental.pallas import tpu as pltpu
from jax.experimental.pallas import tpu_sc as plsc

H = 256
W = 256
N = 8192

TGRID = 8
T = TGRID * TGRID
TPX = (H // TGRID) * (W // TGRID)
K = 2048
GC = 128
NCHUNK = N // GC
NWORK = 32
CPW = NCHUNK // NWORK
TRASH = T * K
PB = 512

RCUT = 4.8995


def _bin_body(m2d_ref, op_ref, col_ref, sc_ref, rot_ref, neg_ref,
              p8_ref, cnt_ref, dst_ref, rad_ref):
    theta = rot_ref[0:1, :] * (2.0 * jnp.pi)
    sx = sc_ref[0:1, :] * 0.02 + 1e-3
    sy = sc_ref[1:2, :] * 0.02 + 1e-3
    ct = jnp.cos(theta)
    st = jnp.sin(theta)
    a = ct * ct * sx * sx + st * st * sy * sy
    b = ct * st * (sx * sx - sy * sy)
    c = st * st * sx * sx + ct * ct * sy * sy
    det = a * c - b * b
    A2 = -0.5 * (c / det)
    B2 = b / det
    C2 = -0.5 * (a / det)
    op = jnp.clip(op_ref[0:1, :], 0.0, 0.99) * neg_ref[0:1, :]
    mx = m2d_ref[0:1, :]
    my = m2d_ref[1:2, :]
    smax = jnp.maximum(sx, sy)
    rad_ref[...] = jnp.ceil(3.0 * smax * float(max(H, W))).astype(jnp.int32)
    p8_ref[...] = jnp.concatenate(
        [mx, my, A2, B2, C2,
         op * col_ref[0:1, :], op * col_ref[1:2, :], op * col_ref[2:3, :]],
        axis=0)

    r = RCUT * smax
    tg = float(TGRID)
    txl = jnp.clip(jnp.floor((mx - r) * tg).astype(jnp.int32), 0, TGRID - 1)
    txh = jnp.clip(jnp.floor((mx + r) * tg).astype(jnp.int32), 0, TGRID - 1)
    tyl = jnp.clip(jnp.floor((my - r) * tg).astype(jnp.int32), 0, TGRID - 1)
    tyh = jnp.clip(jnp.floor((my + r) * tg).astype(jnp.int32), 0, TGRID - 1)

    t2 = lax.broadcasted_iota(jnp.int32, (T, 1), 0)
    t_x = t2 % TGRID
    t_y = t2 // TGRID
    ov = ((t_x >= txl) & (t_x <= txh) & (t_y >= tyl) & (t_y <= tyh))
    ovf = jnp.where(ov, 1.0, 0.0)

    i1 = lax.broadcasted_iota(jnp.int32, (PB, PB), 0)
    j1 = lax.broadcasted_iota(jnp.int32, (PB, PB), 1)
    lt = jnp.where(i1 < j1, 1.0, 0.0)
    offs = jnp.zeros((T, 1), jnp.float32)
    pieces = []
    for blk in range(N // PB):
        sub = ovf[:, blk * PB:(blk + 1) * PB]
        ppos = lax.dot_general(sub, lt, (((1,), (0,)), ((), ())),
                               precision=lax.Precision.HIGHEST,
                               preferred_element_type=jnp.float32) + offs
        pieces.append(ppos)
        offs = ppos[:, PB - 1:PB] + sub[:, PB - 1:PB]
    pos = jnp.concatenate(pieces, axis=1)
    cnt_ref[...] = jnp.minimum(offs, float(K)).astype(jnp.int32)

    posi = pos.astype(jnp.int32)
    for k in range(9):
        ky, kx = k // 3, k % 3
        t_kx = txl + kx
        t_ky = tyl + ky
        t_k = t_ky * TGRID + t_kx
        onehot = t2 == t_k
        sel = jnp.sum(jnp.where(onehot, posi, 0), axis=0, keepdims=True)
        valid = (t_kx <= txh) & (t_ky <= tyh) & (sel < K)
        dst_ref[k:k + 1, :] = jnp.where(valid, t_k * K + sel, TRASH)


def _sc_dispatch_body(p_h, dst_h, tp_h, rows_v, dsts_v, sem):
    cid = lax.axis_index("c")
    sid = lax.axis_index("s")
    wid = sid * 2 + cid
    for c in range(CPW):
        chunk = wid * CPW + c
        pltpu.sync_copy(p_h.at[pl.ds(chunk * GC, GC)], rows_v)
        pltpu.sync_copy(dst_h.at[chunk], dsts_v)
        cps = [pltpu.async_copy(rows_v, tp_h.at[dsts_v.at[k]], sem)
               for k in range(9)]
        for cp in cps:
            cp.wait()


def _raster_body(cnt_sm, tp_ref, img_ref):
    g = pl.program_id(0)
    cnt = cnt_sm[g, 0]
    ty = g // TGRID
    tx = g % TGRID
    i = lax.broadcasted_iota(jnp.int32, (1, TPX), 1)
    tw = W // TGRID
    px = ((tx * tw + (i % tw)).astype(jnp.float32) + 0.5) * (1.0 / W)
    py = ((ty * tw + (i // tw)).astype(jnp.float32) + 0.5) * (1.0 / H)
    rowi = lax.broadcasted_iota(jnp.int32, (GC, 1), 0)

    def chunk(j, acc):
        sl = pl.ds(j * GC, GC)
        mx = tp_ref[0, sl, 0:1]
        my = tp_ref[0, sl, 1:2]
        A2 = tp_ref[0, sl, 2:3]
        B2 = tp_ref[0, sl, 3:4]
        C2 = tp_ref[0, sl, 4:5]
        cT = tp_ref[0, sl, 5:8]
        valid = (j * GC + rowi) < cnt
        cTm = jnp.where(valid, cT, 0.0)
        dx = px - mx
        dy = py - my
        power = A2 * dx * dx + B2 * (dx * dy) + C2 * (dy * dy)
        Gv = jnp.where(valid & (power > -12.0),
                       jnp.exp(jnp.minimum(power, 0.0)), 0.0)
        return acc + lax.dot_general(
            cTm, Gv, (((0,), (0,)), ((), ())),
            precision=lax.Precision.HIGHEST,
            preferred_element_type=jnp.float32)

    nch = (cnt + GC - 1) // GC
    acc = lax.fori_loop(0, nch, chunk, jnp.zeros((3, TPX), jnp.float32))
    img_ref[...] = acc.reshape(1, 3, TPX)


@jax.jit
def kernel(means2D, opacities, colors, scale, rots, negative, bg):
    p8, cnts, dst9, rad = pl.pallas_call(
        _bin_body,
        out_shape=[
            jax.ShapeDtypeStruct((8, N), jnp.float32),
            jax.ShapeDtypeStruct((T, 1), jnp.int32),
            jax.ShapeDtypeStruct((9, N), jnp.int32),
            jax.ShapeDtypeStruct((1, N), jnp.int32),
        ],
    )(means2D.T, opacities.T, colors.T, scale.T, rots.T, negative.T)

    ptab = jnp.pad(p8.T, ((0, 0), (0, 128 - 8)))
    dstr = dst9.reshape(9, NCHUNK, GC).transpose(1, 0, 2)

    mesh = plsc.VectorSubcoreMesh(core_axis_name="c", subcore_axis_name="s",
                                  num_cores=2, num_subcores=16)
    tp_flat, = pl.kernel(
        _sc_dispatch_body,
        out_type=[jax.ShapeDtypeStruct(((T + 1) * K, 128), jnp.float32)],
        mesh=mesh,
        scratch_types=[
            pltpu.VMEM((GC, 128), jnp.float32),
            pltpu.VMEM((9, GC), jnp.int32),
            pltpu.SemaphoreType.DMA,
        ],
    )(ptab, dstr)

    tp3 = tp_flat.reshape(T + 1, K, 128)

    img = pl.pallas_call(
        _raster_body,
        grid=(T,),
        in_specs=[
            pl.BlockSpec(memory_space=pltpu.SMEM),
            pl.BlockSpec((1, K, 128), lambda g: (g, 0, 0)),
        ],
        out_specs=pl.BlockSpec((1, 3, TPX), lambda g: (g, 0, 0)),
        out_shape=jax.ShapeDtypeStruct((T, 3, TPX), jnp.float32),
    )(cnts, tp3)

    tw = W // TGRID
    color = (img.reshape(TGRID, TGRID, 3, tw, tw)
             .transpose(2, 0, 3, 1, 4).reshape(3, H, W) + bg[:, None, None])
    return color, rad.reshape(N)


# speedup vs baseline: 6.7417x; 6.7417x over previous
import jax
import jax.numpy as jnp
from jax import lax
from jax.experimental import pallas as pl
from jax.experimental.pallas import tpu as pltpu
from jax.experimental.pallas import tpu_sc as plsc

H = 256
W = 256
N = 8192

TGRID = 8
T = TGRID * TGRID
TPX = (H // TGRID) * (W // TGRID)
K = 2048
GC = 128
NCHUNK = N // GC
NWORK = 32
CPW = NCHUNK // NWORK
PB = 512

RCUT = 4.8995


def _bin_body(m2d_ref, op_ref, col_ref, sc_ref, rot_ref, neg_ref,
              p8_ref, cnt_ref, dst_ref, rad_ref):
    theta = rot_ref[0:1, :] * (2.0 * jnp.pi)
    sx = sc_ref[0:1, :] * 0.02 + 1e-3
    sy = sc_ref[1:2, :] * 0.02 + 1e-3
    ct = jnp.cos(theta)
    st = jnp.sin(theta)
    a = ct * ct * sx * sx + st * st * sy * sy
    b = ct * st * (sx * sx - sy * sy)
    c = st * st * sx * sx + ct * ct * sy * sy
    det = a * c - b * b
    A2 = -0.5 * (c / det)
    B2 = b / det
    C2 = -0.5 * (a / det)
    op = jnp.clip(op_ref[0:1, :], 0.0, 0.99) * neg_ref[0:1, :]
    mx = m2d_ref[0:1, :]
    my = m2d_ref[1:2, :]
    smax = jnp.maximum(sx, sy)
    rad_ref[...] = jnp.ceil(3.0 * smax * float(max(H, W))).astype(jnp.int32)
    p8_ref[...] = jnp.concatenate(
        [mx, my, A2, B2, C2,
         op * col_ref[0:1, :], op * col_ref[1:2, :], op * col_ref[2:3, :]],
        axis=0)

    r = RCUT * smax
    tg = float(TGRID)
    txl = jnp.clip(jnp.floor((mx - r) * tg).astype(jnp.int32), 0, TGRID - 1)
    txh = jnp.clip(jnp.floor((mx + r) * tg).astype(jnp.int32), 0, TGRID - 1)
    tyl = jnp.clip(jnp.floor((my - r) * tg).astype(jnp.int32), 0, TGRID - 1)
    tyh = jnp.clip(jnp.floor((my + r) * tg).astype(jnp.int32), 0, TGRID - 1)

    t2 = lax.broadcasted_iota(jnp.int32, (T, 1), 0)
    t_x = t2 % TGRID
    t_y = t2 // TGRID
    ov = ((t_x >= txl) & (t_x <= txh) & (t_y >= tyl) & (t_y <= tyh))
    ovf = jnp.where(ov, 1.0, 0.0)

    i1 = lax.broadcasted_iota(jnp.int32, (PB, PB), 0)
    j1 = lax.broadcasted_iota(jnp.int32, (PB, PB), 1)
    lt = jnp.where(i1 < j1, 1.0, 0.0)
    offs = jnp.zeros((T, 1), jnp.float32)
    pieces = []
    for blk in range(N // PB):
        sub = ovf[:, blk * PB:(blk + 1) * PB]
        ppos = lax.dot_general(sub, lt, (((1,), (0,)), ((), ())),
                               precision=lax.Precision.HIGHEST,
                               preferred_element_type=jnp.float32) + offs
        pieces.append(ppos)
        offs = ppos[:, PB - 1:PB] + sub[:, PB - 1:PB]
    pos = jnp.concatenate(pieces, axis=1)
    cnt_ref[...] = jnp.minimum(offs, float(K)).astype(jnp.int32)

    posi = pos.astype(jnp.int32)
    for k in range(9):
        ky, kx = k // 3, k % 3
        t_kx = txl + kx
        t_ky = tyl + ky
        t_k = t_ky * TGRID + t_kx
        onehot = t2 == t_k
        sel = jnp.sum(jnp.where(onehot, posi, 0), axis=0, keepdims=True)
        valid = (t_kx <= txh) & (t_ky <= tyh) & (sel < K)
        gmod = lax.broadcasted_iota(jnp.int32, (1, N), 1) % GC
        trash = T * K + gmod * 9 + k
        dst_ref[k:k + 1, :] = jnp.where(valid, t_k * K + sel, trash)


def _sc_dispatch_body(p_h, dst_h, tp_h, rows_v, dsts_v, sem):
    cid = lax.axis_index("c")
    sid = lax.axis_index("s")
    wid = sid * 2 + cid
    for c in range(CPW):
        chunk = wid * CPW + c
        pltpu.sync_copy(p_h.at[pl.ds(chunk * GC, GC)], rows_v)
        pltpu.sync_copy(dst_h.at[chunk], dsts_v)
        cps = [pltpu.async_copy(rows_v, tp_h.at[dsts_v.at[k]], sem)
               for k in range(9)]
        for cp in cps:
            cp.wait()


def _raster_body(cnt_sm, tp_ref, img_ref):
    g = pl.program_id(0)
    cnt = cnt_sm[g, 0]
    ty = g // TGRID
    tx = g % TGRID
    i = lax.broadcasted_iota(jnp.int32, (1, TPX), 1)
    tw = W // TGRID
    px = ((tx * tw + (i % tw)).astype(jnp.float32) + 0.5) * (1.0 / W)
    py = ((ty * tw + (i // tw)).astype(jnp.float32) + 0.5) * (1.0 / H)
    rowi = lax.broadcasted_iota(jnp.int32, (GC, 1), 0)

    def chunk(j, acc):
        sl = pl.ds(j * GC, GC)
        mx = tp_ref[0, sl, 0:1]
        my = tp_ref[0, sl, 1:2]
        A2 = tp_ref[0, sl, 2:3]
        B2 = tp_ref[0, sl, 3:4]
        C2 = tp_ref[0, sl, 4:5]
        cT = tp_ref[0, sl, 5:8]
        valid = (j * GC + rowi) < cnt
        cTm = jnp.where(valid, cT, 0.0)
        dx = px - mx
        dy = py - my
        power = A2 * dx * dx + B2 * (dx * dy) + C2 * (dy * dy)
        Gv = jnp.where(valid & (power > -12.0),
                       jnp.exp(jnp.minimum(power, 0.0)), 0.0)
        return acc + lax.dot_general(
            cTm, Gv, (((0,), (0,)), ((), ())),
            precision=lax.Precision.HIGHEST,
            preferred_element_type=jnp.float32)

    nch = (cnt + GC - 1) // GC
    acc = lax.fori_loop(0, nch, chunk, jnp.zeros((3, TPX), jnp.float32))
    img_ref[...] = acc.reshape(1, 3, TPX)


@jax.jit
def kernel(means2D, opacities, colors, scale, rots, negative, bg):
    p8, cnts, dst9, rad = pl.pallas_call(
        _bin_body,
        out_shape=[
            jax.ShapeDtypeStruct((8, N), jnp.float32),
            jax.ShapeDtypeStruct((T, 1), jnp.int32),
            jax.ShapeDtypeStruct((9, N), jnp.int32),
            jax.ShapeDtypeStruct((1, N), jnp.int32),
        ],
    )(means2D.T, opacities.T, colors.T, scale.T, rots.T, negative.T)

    ptab = jnp.pad(p8.T, ((0, 0), (0, 128 - 8)))
    dstr = dst9.reshape(9, NCHUNK, GC).transpose(1, 0, 2)

    mesh = plsc.VectorSubcoreMesh(core_axis_name="c", subcore_axis_name="s",
                                  num_cores=2, num_subcores=16)
    tp_flat, = pl.kernel(
        _sc_dispatch_body,
        out_type=[jax.ShapeDtypeStruct(((T + 1) * K, 128), jnp.float32)],
        mesh=mesh,
        scratch_types=[
            pltpu.VMEM((GC, 128), jnp.float32),
            pltpu.VMEM((9, GC), jnp.int32),
            pltpu.SemaphoreType.DMA,
        ],
    )(ptab, dstr)

    tp3 = tp_flat.reshape(T + 1, K, 128)

    img = pl.pallas_call(
        _raster_body,
        grid=(T,),
        in_specs=[
            pl.BlockSpec(memory_space=pltpu.SMEM),
            pl.BlockSpec((1, K, 128), lambda g: (g, 0, 0)),
        ],
        out_specs=pl.BlockSpec((1, 3, TPX), lambda g: (g, 0, 0)),
        out_shape=jax.ShapeDtypeStruct((T, 3, TPX), jnp.float32),
    )(cnts, tp3)

    tw = W // TGRID
    color = (img.reshape(TGRID, TGRID, 3, tw, tw)
             .transpose(2, 0, 3, 1, 4).reshape(3, H, W) + bg[:, None, None])
    return color, rad.reshape(N)


# speedup vs baseline: 6.9462x; 1.0303x over previous
import jax
import jax.numpy as jnp
from jax import lax
from jax.experimental import pallas as pl
from jax.experimental.pallas import tpu as pltpu
from jax.experimental.pallas import tpu_sc as plsc

H = 256
W = 256
N = 8192

TGRID = 8
T = TGRID * TGRID
TPX = (H // TGRID) * (W // TGRID)
K = 2048
GC = 128
NCHUNK = N // GC
NWORK = 32
CPW = NCHUNK // NWORK
PB = 512

RCUT = 4.8995


def _bin_body(m2d_ref, op_ref, col_ref, sc_ref, rot_ref, neg_ref,
              p8_ref, cnt_ref, dst_ref, rad_ref):
    theta = rot_ref[0:1, :] * (2.0 * jnp.pi)
    sx = sc_ref[0:1, :] * 0.02 + 1e-3
    sy = sc_ref[1:2, :] * 0.02 + 1e-3
    ct = jnp.cos(theta)
    st = jnp.sin(theta)
    a = ct * ct * sx * sx + st * st * sy * sy
    b = ct * st * (sx * sx - sy * sy)
    c = st * st * sx * sx + ct * ct * sy * sy
    det = a * c - b * b
    A2 = -0.5 * (c / det)
    B2 = b / det
    C2 = -0.5 * (a / det)
    op = jnp.clip(op_ref[0:1, :], 0.0, 0.99) * neg_ref[0:1, :]
    mx = m2d_ref[0:1, :]
    my = m2d_ref[1:2, :]
    smax = jnp.maximum(sx, sy)
    rad_ref[...] = jnp.ceil(3.0 * smax * float(max(H, W))).astype(jnp.int32)
    p8_ref[...] = jnp.concatenate(
        [mx, my, A2, B2, C2,
         op * col_ref[0:1, :], op * col_ref[1:2, :], op * col_ref[2:3, :]],
        axis=0)

    r = RCUT * smax
    tg = float(TGRID)
    txl = jnp.clip(jnp.floor((mx - r) * tg).astype(jnp.int32), 0, TGRID - 1)
    txh = jnp.clip(jnp.floor((mx + r) * tg).astype(jnp.int32), 0, TGRID - 1)
    tyl = jnp.clip(jnp.floor((my - r) * tg).astype(jnp.int32), 0, TGRID - 1)
    tyh = jnp.clip(jnp.floor((my + r) * tg).astype(jnp.int32), 0, TGRID - 1)

    t2 = lax.broadcasted_iota(jnp.int32, (T, 1), 0)
    t_x = t2 % TGRID
    t_y = t2 // TGRID
    ov = ((t_x >= txl) & (t_x <= txh) & (t_y >= tyl) & (t_y <= tyh))
    ovf = jnp.where(ov, 1.0, 0.0)

    i1 = lax.broadcasted_iota(jnp.int32, (PB, PB), 0)
    j1 = lax.broadcasted_iota(jnp.int32, (PB, PB), 1)
    lt = jnp.where(i1 < j1, 1.0, 0.0)
    offs = jnp.zeros((T, 1), jnp.float32)
    pieces = []
    for blk in range(N // PB):
        sub = ovf[:, blk * PB:(blk + 1) * PB]
        ppos = lax.dot_general(sub, lt, (((1,), (0,)), ((), ())),
                               precision=lax.Precision.HIGHEST,
                               preferred_element_type=jnp.float32) + offs
        pieces.append(ppos)
        offs = ppos[:, PB - 1:PB] + sub[:, PB - 1:PB]
    pos = jnp.concatenate(pieces, axis=1)
    cnt_ref[...] = jnp.minimum(offs, float(K)).astype(jnp.int32)

    posi = pos.astype(jnp.int32)
    for k in range(9):
        ky, kx = k // 3, k % 3
        t_kx = txl + kx
        t_ky = tyl + ky
        t_k = t_ky * TGRID + t_kx
        onehot = t2 == t_k
        sel = jnp.sum(jnp.where(onehot, posi, 0), axis=0, keepdims=True)
        valid = (t_kx <= txh) & (t_ky <= tyh) & (sel < K)
        gmod = lax.broadcasted_iota(jnp.int32, (1, N), 1) % GC
        trash = T * K + gmod * 9 + k
        dst_ref[k:k + 1, :] = jnp.where(valid, t_k * K + sel, trash)


def _sc_dispatch_body(p_h, dst_h, tp_h, rows_v, dsts_v, sem):
    cid = lax.axis_index("c")
    sid = lax.axis_index("s")
    wid = sid * 2 + cid
    for c in range(CPW):
        chunk = wid * CPW + c
        pltpu.sync_copy(p_h.at[pl.ds(chunk * GC, GC)], rows_v)
        pltpu.sync_copy(dst_h.at[chunk], dsts_v)
        cps = [pltpu.async_copy(rows_v, tp_h.at[dsts_v.at[k]], sem)
               for k in range(9)]
        for cp in cps:
            cp.wait()


RGC = 256


def _raster_body(cnt_sm, tp_ref, img_ref):
    g = pl.program_id(0)
    cnt = cnt_sm[g, 0]
    ty = g // TGRID
    tx = g % TGRID
    i = lax.broadcasted_iota(jnp.int32, (1, TPX), 1)
    tw = W // TGRID
    px = ((tx * tw + (i % tw)).astype(jnp.float32) + 0.5) * (1.0 / W)
    py = ((ty * tw + (i // tw)).astype(jnp.float32) + 0.5) * (1.0 / H)
    rowi = lax.broadcasted_iota(jnp.int32, (RGC, 1), 0)

    def chunk(j, acc):
        sl = pl.ds(j * RGC, RGC)
        mx = tp_ref[0, sl, 0:1]
        my = tp_ref[0, sl, 1:2]
        A2 = tp_ref[0, sl, 2:3]
        B2 = tp_ref[0, sl, 3:4]
        C2 = tp_ref[0, sl, 4:5]
        cT = tp_ref[0, sl, 5:8]
        valid = (j * RGC + rowi) < cnt
        cTm = jnp.where(valid, cT, 0.0)
        dx = px - mx
        dy = py - my
        power = A2 * dx * dx + B2 * (dx * dy) + C2 * (dy * dy)
        Gv = jnp.where(valid & (power > -12.0),
                       jnp.exp(jnp.minimum(power, 0.0)), 0.0)
        return acc + lax.dot_general(
            cTm, Gv, (((0,), (0,)), ((), ())),
            precision=lax.Precision.HIGHEST,
            preferred_element_type=jnp.float32)

    nch = (cnt + RGC - 1) // RGC
    acc = lax.fori_loop(0, nch, chunk, jnp.zeros((3, TPX), jnp.float32))
    img_ref[...] = acc.reshape(1, 3, TPX)


@jax.jit
def kernel(means2D, opacities, colors, scale, rots, negative, bg):
    p8, cnts, dst9, rad = pl.pallas_call(
        _bin_body,
        out_shape=[
            jax.ShapeDtypeStruct((8, N), jnp.float32),
            jax.ShapeDtypeStruct((T, 1), jnp.int32),
            jax.ShapeDtypeStruct((9, N), jnp.int32),
            jax.ShapeDtypeStruct((1, N), jnp.int32),
        ],
    )(means2D.T, opacities.T, colors.T, scale.T, rots.T, negative.T)

    ptab = jnp.pad(p8.T, ((0, 0), (0, 128 - 8)))
    dstr = dst9.reshape(9, NCHUNK, GC).transpose(1, 0, 2)

    mesh = plsc.VectorSubcoreMesh(core_axis_name="c", subcore_axis_name="s",
                                  num_cores=2, num_subcores=16)
    tp_flat, = pl.kernel(
        _sc_dispatch_body,
        out_type=[jax.ShapeDtypeStruct(((T + 1) * K, 128), jnp.float32)],
        mesh=mesh,
        scratch_types=[
            pltpu.VMEM((GC, 128), jnp.float32),
            pltpu.VMEM((9, GC), jnp.int32),
            pltpu.SemaphoreType.DMA,
        ],
    )(ptab, dstr)

    tp3 = tp_flat.reshape(T + 1, K, 128)

    img = pl.pallas_call(
        _raster_body,
        grid=(T,),
        in_specs=[
            pl.BlockSpec(memory_space=pltpu.SMEM),
            pl.BlockSpec((1, K, 128), lambda g: (g, 0, 0)),
        ],
        out_specs=pl.BlockSpec((1, 3, TPX), lambda g: (g, 0, 0)),
        out_shape=jax.ShapeDtypeStruct((T, 3, TPX), jnp.float32),
    )(cnts, tp3)

    tw = W // TGRID
    color = (img.reshape(TGRID, TGRID, 3, tw, tw)
             .transpose(2, 0, 3, 1, 4).reshape(3, H, W) + bg[:, None, None])
    return color, rad.reshape(N)


# speedup vs baseline: 7.0370x; 1.0131x over previous
import jax
import jax.numpy as jnp
from jax import lax
from jax.experimental import pallas as pl
from jax.experimental.pallas import tpu as pltpu
from jax.experimental.pallas import tpu_sc as plsc

H = 256
W = 256
N = 8192

TGRID = 8
T = TGRID * TGRID
TPX = (H // TGRID) * (W // TGRID)
K = 2048
GC = 128
NCHUNK = N // GC
NWORK = 32
CPW = NCHUNK // NWORK
PB = 512

RCUT = 4.8995


def _bin_body(m2d_ref, op_ref, col_ref, sc_ref, rot_ref, neg_ref,
              p8_ref, cnt_ref, dst_ref, rad_ref):
    theta = rot_ref[0:1, :] * (2.0 * jnp.pi)
    sx = sc_ref[0:1, :] * 0.02 + 1e-3
    sy = sc_ref[1:2, :] * 0.02 + 1e-3
    ct = jnp.cos(theta)
    st = jnp.sin(theta)
    a = ct * ct * sx * sx + st * st * sy * sy
    b = ct * st * (sx * sx - sy * sy)
    c = st * st * sx * sx + ct * ct * sy * sy
    det = a * c - b * b
    A2 = -0.5 * (c / det)
    B2 = b / det
    C2 = -0.5 * (a / det)
    op = jnp.clip(op_ref[0:1, :], 0.0, 0.99) * neg_ref[0:1, :]
    mx = m2d_ref[0:1, :]
    my = m2d_ref[1:2, :]
    smax = jnp.maximum(sx, sy)
    rad_ref[...] = jnp.ceil(3.0 * smax * float(max(H, W))).astype(jnp.int32)
    p8_ref[...] = jnp.concatenate(
        [mx, my, A2, B2, C2,
         op * col_ref[0:1, :], op * col_ref[1:2, :], op * col_ref[2:3, :]],
        axis=0)

    r = RCUT * smax
    tg = float(TGRID)
    txl = jnp.clip(jnp.floor((mx - r) * tg).astype(jnp.int32), 0, TGRID - 1)
    txh = jnp.clip(jnp.floor((mx + r) * tg).astype(jnp.int32), 0, TGRID - 1)
    tyl = jnp.clip(jnp.floor((my - r) * tg).astype(jnp.int32), 0, TGRID - 1)
    tyh = jnp.clip(jnp.floor((my + r) * tg).astype(jnp.int32), 0, TGRID - 1)

    t2 = lax.broadcasted_iota(jnp.int32, (T, 1), 0)
    t_x = t2 % TGRID
    t_y = t2 // TGRID
    ov = ((t_x >= txl) & (t_x <= txh) & (t_y >= tyl) & (t_y <= tyh))
    ovf = jnp.where(ov, 1.0, 0.0)

    i1 = lax.broadcasted_iota(jnp.int32, (PB, PB), 0)
    j1 = lax.broadcasted_iota(jnp.int32, (PB, PB), 1)
    lt = jnp.where(i1 < j1, 1.0, 0.0)
    offs = jnp.zeros((T, 1), jnp.float32)
    pieces = []
    for blk in range(N // PB):
        sub = ovf[:, blk * PB:(blk + 1) * PB]
        ppos = lax.dot_general(sub, lt, (((1,), (0,)), ((), ())),
                               precision=lax.Precision.HIGHEST,
                               preferred_element_type=jnp.float32) + offs
        pieces.append(ppos)
        offs = ppos[:, PB - 1:PB] + sub[:, PB - 1:PB]
    pos = jnp.concatenate(pieces, axis=1)
    cnt_ref[...] = jnp.minimum(offs, float(K)).astype(jnp.int32)

    posi = pos.astype(jnp.int32)
    for k in range(9):
        ky, kx = k // 3, k % 3
        t_kx = txl + kx
        t_ky = tyl + ky
        t_k = t_ky * TGRID + t_kx
        onehot = t2 == t_k
        sel = jnp.sum(jnp.where(onehot, posi, 0), axis=0, keepdims=True)
        valid = (t_kx <= txh) & (t_ky <= tyh) & (sel < K)
        gmod = lax.broadcasted_iota(jnp.int32, (1, N), 1) % GC
        trash = T * K + gmod * 9 + k
        dst_ref[k:k + 1, :] = jnp.where(valid, t_k * K + sel, trash)


def _sc_dispatch_body(p_h, dst_h, tp_h, rows_v, dsts_v, sem):
    cid = lax.axis_index("c")
    sid = lax.axis_index("s")
    wid = sid * 2 + cid
    for c in range(CPW):
        chunk = wid * CPW + c
        pltpu.sync_copy(p_h.at[pl.ds(chunk * GC, GC)], rows_v)
        pltpu.sync_copy(dst_h.at[chunk], dsts_v)
        cps = [pltpu.async_copy(rows_v, tp_h.at[dsts_v.at[k]], sem)
               for k in range(9)]
        for cp in cps:
            cp.wait()


RGC = 256


def _raster_body(cnt_sm, tp_ref, img_ref):
    g = pl.program_id(0)
    cnt = cnt_sm[g, 0]
    ty = g // TGRID
    tx = g % TGRID
    i = lax.broadcasted_iota(jnp.int32, (1, TPX), 1)
    tw = W // TGRID
    px = ((tx * tw + (i % tw)).astype(jnp.float32) + 0.5) * (1.0 / W)
    py = ((ty * tw + (i // tw)).astype(jnp.float32) + 0.5) * (1.0 / H)
    rowi = lax.broadcasted_iota(jnp.int32, (RGC, 1), 0)

    def chunk(j, acc):
        sl = pl.ds(j * RGC, RGC)
        mx = tp_ref[0, sl, 0:1]
        my = tp_ref[0, sl, 1:2]
        A2 = tp_ref[0, sl, 2:3]
        B2 = tp_ref[0, sl, 3:4]
        C2 = tp_ref[0, sl, 4:5]
        cT = tp_ref[0, sl, 5:8]
        valid = (j * RGC + rowi) < cnt
        cTm = jnp.where(valid, cT, 0.0)
        thr = jnp.where(valid, -12.0, jnp.inf)
        dx = px - mx
        dy = py - my
        power = A2 * dx * dx + B2 * (dx * dy) + C2 * (dy * dy)
        Gv = jnp.where(power > thr, jnp.exp(power), 0.0)
        return acc + lax.dot_general(
            cTm, Gv, (((0,), (0,)), ((), ())),
            precision=lax.Precision.HIGHEST,
            preferred_element_type=jnp.float32)

    nch = (cnt + RGC - 1) // RGC
    acc = lax.fori_loop(0, nch, chunk, jnp.zeros((3, TPX), jnp.float32))
    img_ref[...] = acc.reshape(1, 3, TPX)


@jax.jit
def kernel(means2D, opacities, colors, scale, rots, negative, bg):
    p8, cnts, dst9, rad = pl.pallas_call(
        _bin_body,
        out_shape=[
            jax.ShapeDtypeStruct((8, N), jnp.float32),
            jax.ShapeDtypeStruct((T, 1), jnp.int32),
            jax.ShapeDtypeStruct((9, N), jnp.int32),
            jax.ShapeDtypeStruct((1, N), jnp.int32),
        ],
    )(means2D.T, opacities.T, colors.T, scale.T, rots.T, negative.T)

    ptab = jnp.pad(p8.T, ((0, 0), (0, 128 - 8)))
    dstr = dst9.reshape(9, NCHUNK, GC).transpose(1, 0, 2)

    mesh = plsc.VectorSubcoreMesh(core_axis_name="c", subcore_axis_name="s",
                                  num_cores=2, num_subcores=16)
    tp_flat, = pl.kernel(
        _sc_dispatch_body,
        out_type=[jax.ShapeDtypeStruct(((T + 1) * K, 128), jnp.float32)],
        mesh=mesh,
        scratch_types=[
            pltpu.VMEM((GC, 128), jnp.float32),
            pltpu.VMEM((9, GC), jnp.int32),
            pltpu.SemaphoreType.DMA,
        ],
    )(ptab, dstr)

    tp3 = tp_flat.reshape(T + 1, K, 128)

    img = pl.pallas_call(
        _raster_body,
        grid=(T,),
        in_specs=[
            pl.BlockSpec(memory_space=pltpu.SMEM),
            pl.BlockSpec((1, K, 128), lambda g: (g, 0, 0)),
        ],
        out_specs=pl.BlockSpec((1, 3, TPX), lambda g: (g, 0, 0)),
        out_shape=jax.ShapeDtypeStruct((T, 3, TPX), jnp.float32),
    )(cnts, tp3)

    tw = W // TGRID
    color = (img.reshape(TGRID, TGRID, 3, tw, tw)
             .transpose(2, 0, 3, 1, 4).reshape(3, H, W) + bg[:, None, None])
    return color, rad.reshape(N)


# speedup vs baseline: 8.4386x; 1.1992x over previous
import jax
import jax.numpy as jnp
from jax import lax
from jax.experimental import pallas as pl
from jax.experimental.pallas import tpu as pltpu
from jax.experimental.pallas import tpu_sc as plsc

H = 256
W = 256
N = 8192

TGRID = 8
T = TGRID * TGRID
TPX = (H // TGRID) * (W // TGRID)
K = 2048
GC = 128
NCHUNK = N // GC
NWORK = 32
CPW = NCHUNK // NWORK
PB = 512

RCUT = 4.8995


def _bin_body(m2d_ref, op_ref, col_ref, sc_ref, rot_ref, neg_ref,
              p8_ref, cnt_ref, dst_ref, rad_ref):
    theta = rot_ref[0:1, :] * (2.0 * jnp.pi)
    sx = sc_ref[0:1, :] * 0.02 + 1e-3
    sy = sc_ref[1:2, :] * 0.02 + 1e-3
    ct = jnp.cos(theta)
    st = jnp.sin(theta)
    a = ct * ct * sx * sx + st * st * sy * sy
    b = ct * st * (sx * sx - sy * sy)
    c = st * st * sx * sx + ct * ct * sy * sy
    det = a * c - b * b
    A2 = -0.5 * (c / det)
    B2 = b / det
    C2 = -0.5 * (a / det)
    op = jnp.clip(op_ref[0:1, :], 0.0, 0.99) * neg_ref[0:1, :]
    mx = m2d_ref[0:1, :]
    my = m2d_ref[1:2, :]
    smax = jnp.maximum(sx, sy)
    rad_ref[...] = jnp.ceil(3.0 * smax * float(max(H, W))).astype(jnp.int32)
    p8_ref[...] = jnp.concatenate(
        [mx, my, A2, B2, C2,
         op * col_ref[0:1, :], op * col_ref[1:2, :], op * col_ref[2:3, :]],
        axis=0)

    rx = RCUT * jnp.sqrt(a)
    ry = RCUT * jnp.sqrt(c)
    tg = float(TGRID)
    txl = jnp.clip(jnp.floor((mx - rx) * tg).astype(jnp.int32), 0, TGRID - 1)
    txh = jnp.clip(jnp.floor((mx + rx) * tg).astype(jnp.int32), 0, TGRID - 1)
    tyl = jnp.clip(jnp.floor((my - ry) * tg).astype(jnp.int32), 0, TGRID - 1)
    tyh = jnp.clip(jnp.floor((my + ry) * tg).astype(jnp.int32), 0, TGRID - 1)

    t2 = lax.broadcasted_iota(jnp.int32, (T, 1), 0)
    t_x = t2 % TGRID
    t_y = t2 // TGRID
    ov = ((t_x >= txl) & (t_x <= txh) & (t_y >= tyl) & (t_y <= tyh))
    ovf = jnp.where(ov, 1.0, 0.0)

    i1 = lax.broadcasted_iota(jnp.int32, (PB, PB), 0)
    j1 = lax.broadcasted_iota(jnp.int32, (PB, PB), 1)
    lt = jnp.where(i1 < j1, 1.0, 0.0)
    offs = jnp.zeros((T, 1), jnp.float32)
    pieces = []
    for blk in range(N // PB):
        sub = ovf[:, blk * PB:(blk + 1) * PB]
        ppos = lax.dot_general(sub, lt, (((1,), (0,)), ((), ())),
                               precision=lax.Precision.HIGHEST,
                               preferred_element_type=jnp.float32) + offs
        pieces.append(ppos)
        offs = ppos[:, PB - 1:PB] + sub[:, PB - 1:PB]
    pos = jnp.concatenate(pieces, axis=1)
    cnt_ref[...] = jnp.minimum(offs, float(K)).astype(jnp.int32)

    posi = pos.astype(jnp.int32)
    for k in range(9):
        ky, kx = k // 3, k % 3
        t_kx = txl + kx
        t_ky = tyl + ky
        t_k = t_ky * TGRID + t_kx
        onehot = t2 == t_k
        sel = jnp.sum(jnp.where(onehot, posi, 0), axis=0, keepdims=True)
        valid = (t_kx <= txh) & (t_ky <= tyh) & (sel < K)
        gmod = lax.broadcasted_iota(jnp.int32, (1, N), 1) % GC
        trash = T * K + gmod * 9 + k
        dst_ref[k:k + 1, :] = jnp.where(valid, t_k * K + sel, trash)


def _sc_dispatch_body(p_h, dst_h, tp_h, rows_v, dsts_v, sem):
    cid = lax.axis_index("c")
    sid = lax.axis_index("s")
    wid = sid * 2 + cid
    for c in range(CPW):
        chunk = wid * CPW + c
        pltpu.sync_copy(p_h.at[pl.ds(chunk * GC, GC)], rows_v)
        pltpu.sync_copy(dst_h.at[chunk], dsts_v)
        cps = [pltpu.async_copy(rows_v, tp_h.at[dsts_v.at[k]], sem)
               for k in range(9)]
        for cp in cps:
            cp.wait()


RGC = 256


def _raster_body(cnt_sm, tp_ref, img_ref):
    g = pl.program_id(0)
    cnt = cnt_sm[g, 0]
    ty = g // TGRID
    tx = g % TGRID
    i = lax.broadcasted_iota(jnp.int32, (1, TPX), 1)
    tw = W // TGRID
    px = ((tx * tw + (i % tw)).astype(jnp.float32) + 0.5) * (1.0 / W)
    py = ((ty * tw + (i // tw)).astype(jnp.float32) + 0.5) * (1.0 / H)
    rowi = lax.broadcasted_iota(jnp.int32, (RGC, 1), 0)

    def chunk(j, acc):
        sl = pl.ds(j * RGC, RGC)
        mx = tp_ref[0, sl, 0:1]
        my = tp_ref[0, sl, 1:2]
        A2 = tp_ref[0, sl, 2:3]
        B2 = tp_ref[0, sl, 3:4]
        C2 = tp_ref[0, sl, 4:5]
        cT = tp_ref[0, sl, 5:8]
        valid = (j * RGC + rowi) < cnt
        cTm = jnp.where(valid, cT, 0.0)
        thr = jnp.where(valid, -12.0, jnp.inf)
        dx = px - mx
        dy = py - my
        power = dx * (A2 * dx + B2 * dy) + C2 * (dy * dy)
        Gv = jnp.where(power > thr, jnp.exp(power), 0.0)
        return acc + lax.dot_general(
            cTm, Gv, (((0,), (0,)), ((), ())),
            precision=lax.Precision.HIGHEST,
            preferred_element_type=jnp.float32)

    nch = (cnt + RGC - 1) // RGC
    acc = lax.fori_loop(0, nch, chunk, jnp.zeros((3, TPX), jnp.float32))
    img_ref[...] = acc.reshape(1, 3, TPX)


@jax.jit
def kernel(means2D, opacities, colors, scale, rots, negative, bg):
    p8, cnts, dst9, rad = pl.pallas_call(
        _bin_body,
        out_shape=[
            jax.ShapeDtypeStruct((8, N), jnp.float32),
            jax.ShapeDtypeStruct((T, 1), jnp.int32),
            jax.ShapeDtypeStruct((9, N), jnp.int32),
            jax.ShapeDtypeStruct((1, N), jnp.int32),
        ],
    )(means2D.T, opacities.T, colors.T, scale.T, rots.T, negative.T)

    ptab = jnp.pad(p8.T, ((0, 0), (0, 128 - 8)))
    dstr = dst9.reshape(9, NCHUNK, GC).transpose(1, 0, 2)

    mesh = plsc.VectorSubcoreMesh(core_axis_name="c", subcore_axis_name="s",
                                  num_cores=2, num_subcores=16)
    tp_flat, = pl.kernel(
        _sc_dispatch_body,
        out_type=[jax.ShapeDtypeStruct(((T + 1) * K, 128), jnp.float32)],
        mesh=mesh,
        scratch_types=[
            pltpu.VMEM((GC, 128), jnp.float32),
            pltpu.VMEM((9, GC), jnp.int32),
            pltpu.SemaphoreType.DMA,
        ],
    )(ptab, dstr)

    tp3 = tp_flat.reshape(T + 1, K, 128)

    img = pl.pallas_call(
        _raster_body,
        grid=(T,),
        in_specs=[
            pl.BlockSpec(memory_space=pltpu.SMEM),
            pl.BlockSpec((1, K, 128), lambda g: (g, 0, 0)),
        ],
        out_specs=pl.BlockSpec((1, 3, TPX), lambda g: (g, 0, 0)),
        out_shape=jax.ShapeDtypeStruct((T, 3, TPX), jnp.float32),
    )(cnts, tp3)

    tw = W // TGRID
    color = (img.reshape(TGRID, TGRID, 3, tw, tw)
             .transpose(2, 0, 3, 1, 4).reshape(3, H, W) + bg[:, None, None])
    return color, rad.reshape(N)


# speedup vs baseline: 10.9776x; 1.3009x over previous
import jax
import jax.numpy as jnp
from jax import lax
from jax.experimental import pallas as pl
from jax.experimental.pallas import tpu as pltpu
from jax.experimental.pallas import tpu_sc as plsc

H = 256
W = 256
N = 8192

TGRID = 8
T = TGRID * TGRID
TPX = (H // TGRID) * (W // TGRID)
K = 2048
GC = 128
NCHUNK = N // GC
NWORK = 32
CPW = NCHUNK // NWORK
PB = 512

RCUT = 4.8995


def _bin_body(m2d_ref, op_ref, col_ref, sc_ref, rot_ref, neg_ref,
              p8_ref, cnt_ref, dst_ref, rad_ref):
    theta = rot_ref[0:1, :] * (2.0 * jnp.pi)
    sx = sc_ref[0:1, :] * 0.02 + 1e-3
    sy = sc_ref[1:2, :] * 0.02 + 1e-3
    ct = jnp.cos(theta)
    st = jnp.sin(theta)
    a = ct * ct * sx * sx + st * st * sy * sy
    b = ct * st * (sx * sx - sy * sy)
    c = st * st * sx * sx + ct * ct * sy * sy
    det = a * c - b * b
    A2 = -0.5 * (c / det)
    B2 = b / det
    C2 = -0.5 * (a / det)
    op = jnp.clip(op_ref[0:1, :], 0.0, 0.99) * neg_ref[0:1, :]
    mx = m2d_ref[0:1, :]
    my = m2d_ref[1:2, :]
    smax = jnp.maximum(sx, sy)
    rad_ref[...] = jnp.ceil(3.0 * smax * float(max(H, W))).astype(jnp.int32)
    p8_ref[...] = jnp.concatenate(
        [mx, my, A2, B2, C2,
         op * col_ref[0:1, :], op * col_ref[1:2, :], op * col_ref[2:3, :]],
        axis=0)

    rx = RCUT * jnp.sqrt(a)
    ry = RCUT * jnp.sqrt(c)
    tg = float(TGRID)
    txl = jnp.clip(jnp.floor((mx - rx) * tg).astype(jnp.int32), 0, TGRID - 1)
    txh = jnp.clip(jnp.floor((mx + rx) * tg).astype(jnp.int32), 0, TGRID - 1)
    tyl = jnp.clip(jnp.floor((my - ry) * tg).astype(jnp.int32), 0, TGRID - 1)
    tyh = jnp.clip(jnp.floor((my + ry) * tg).astype(jnp.int32), 0, TGRID - 1)

    t2 = lax.broadcasted_iota(jnp.int32, (T, 1), 0)
    t_x = t2 % TGRID
    t_y = t2 // TGRID
    ov = ((t_x >= txl) & (t_x <= txh) & (t_y >= tyl) & (t_y <= tyh))
    ovf = jnp.where(ov, 1.0, 0.0)

    i1 = lax.broadcasted_iota(jnp.int32, (PB, PB), 0)
    j1 = lax.broadcasted_iota(jnp.int32, (PB, PB), 1)
    lt = jnp.where(i1 < j1, 1.0, 0.0)
    offs = jnp.zeros((T, 1), jnp.float32)
    pieces = []
    for blk in range(N // PB):
        sub = ovf[:, blk * PB:(blk + 1) * PB]
        ppos = lax.dot_general(sub, lt, (((1,), (0,)), ((), ())),
                               precision=lax.Precision.HIGHEST,
                               preferred_element_type=jnp.float32) + offs
        pieces.append(ppos)
        offs = ppos[:, PB - 1:PB] + sub[:, PB - 1:PB]
    pos = jnp.concatenate(pieces, axis=1)
    cnt_ref[...] = jnp.minimum(offs, float(K)).astype(jnp.int32)

    posi = pos.astype(jnp.int32)
    for k in range(9):
        ky, kx = k // 3, k % 3
        t_kx = txl + kx
        t_ky = tyl + ky
        t_k = t_ky * TGRID + t_kx
        onehot = t2 == t_k
        sel = jnp.sum(jnp.where(onehot, posi, 0), axis=0, keepdims=True)
        valid = (t_kx <= txh) & (t_ky <= tyh) & (sel < K)
        gmod = lax.broadcasted_iota(jnp.int32, (1, N), 1) % GC
        trash = T * K + gmod * 9 + k
        dst_ref[k:k + 1, :] = jnp.where(valid, t_k * K + sel, trash)


def _sc_dispatch_body(p_h, dst_h, tp_h, rows_v, dsts_v, sem):
    cid = lax.axis_index("c")
    sid = lax.axis_index("s")
    wid = sid * 2 + cid
    for c in range(CPW):
        chunk = wid * CPW + c
        pltpu.sync_copy(p_h.at[pl.ds(chunk * GC, GC)], rows_v)
        pltpu.sync_copy(dst_h.at[chunk], dsts_v)
        cps = [pltpu.async_copy(rows_v, tp_h.at[dsts_v.at[k]], sem)
               for k in range(9)]
        for cp in cps:
            cp.wait()


RGC = 256


def _raster_body(cnt_sm, tp_ref, img_ref):
    g = pl.program_id(0)
    cnt = cnt_sm[g, 0]
    ty = g // TGRID
    tx = g % TGRID
    i = lax.broadcasted_iota(jnp.int32, (1, TPX), 1)
    tw = W // TGRID
    px = ((tx * tw + (i % tw)).astype(jnp.float32) + 0.5) * (1.0 / W)
    py = ((ty * tw + (i // tw)).astype(jnp.float32) + 0.5) * (1.0 / H)
    rowi = lax.broadcasted_iota(jnp.int32, (RGC, 1), 0)

    def chunk(j, acc):
        sl = pl.ds(j * RGC, RGC)
        mx = tp_ref[0, sl, 0:1]
        my = tp_ref[0, sl, 1:2]
        A2 = tp_ref[0, sl, 2:3]
        B2 = tp_ref[0, sl, 3:4]
        C2 = tp_ref[0, sl, 4:5]
        cT = tp_ref[0, sl, 5:8]
        valid = (j * RGC + rowi) < cnt
        cTm = jnp.where(valid, cT, 0.0)
        thr = jnp.where(valid, -12.0, jnp.inf)
        dx = px - mx
        dy = py - my
        power = dx * (A2 * dx + B2 * dy) + C2 * (dy * dy)
        Gv = jnp.where(power > thr, jnp.exp(power), 0.0)
        return acc + lax.dot_general(
            cTm, Gv, (((0,), (0,)), ((), ())),
            preferred_element_type=jnp.float32)

    nch = (cnt + RGC - 1) // RGC
    acc = lax.fori_loop(0, nch, chunk, jnp.zeros((3, TPX), jnp.float32))
    img_ref[...] = acc.reshape(1, 3, TPX)


@jax.jit
def kernel(means2D, opacities, colors, scale, rots, negative, bg):
    p8, cnts, dst9, rad = pl.pallas_call(
        _bin_body,
        out_shape=[
            jax.ShapeDtypeStruct((8, N), jnp.float32),
            jax.ShapeDtypeStruct((T, 1), jnp.int32),
            jax.ShapeDtypeStruct((9, N), jnp.int32),
            jax.ShapeDtypeStruct((1, N), jnp.int32),
        ],
    )(means2D.T, opacities.T, colors.T, scale.T, rots.T, negative.T)

    ptab = jnp.pad(p8.T, ((0, 0), (0, 128 - 8)))
    dstr = dst9.reshape(9, NCHUNK, GC).transpose(1, 0, 2)

    mesh = plsc.VectorSubcoreMesh(core_axis_name="c", subcore_axis_name="s",
                                  num_cores=2, num_subcores=16)
    tp_flat, = pl.kernel(
        _sc_dispatch_body,
        out_type=[jax.ShapeDtypeStruct(((T + 1) * K, 128), jnp.float32)],
        mesh=mesh,
        scratch_types=[
            pltpu.VMEM((GC, 128), jnp.float32),
            pltpu.VMEM((9, GC), jnp.int32),
            pltpu.SemaphoreType.DMA,
        ],
    )(ptab, dstr)

    tp3 = tp_flat.reshape(T + 1, K, 128)

    img = pl.pallas_call(
        _raster_body,
        grid=(T,),
        in_specs=[
            pl.BlockSpec(memory_space=pltpu.SMEM),
            pl.BlockSpec((1, K, 128), lambda g: (g, 0, 0)),
        ],
        out_specs=pl.BlockSpec((1, 3, TPX), lambda g: (g, 0, 0)),
        out_shape=jax.ShapeDtypeStruct((T, 3, TPX), jnp.float32),
    )(cnts, tp3)

    tw = W // TGRID
    color = (img.reshape(TGRID, TGRID, 3, tw, tw)
             .transpose(2, 0, 3, 1, 4).reshape(3, H, W) + bg[:, None, None])
    return color, rad.reshape(N)


# speedup vs baseline: 11.0954x; 1.0107x over previous
import jax
import jax.numpy as jnp
from jax import lax
from jax.experimental import pallas as pl
from jax.experimental.pallas import tpu as pltpu
from jax.experimental.pallas import tpu_sc as plsc

H = 256
W = 256
N = 8192

TGRID = 8
T = TGRID * TGRID
TPX = (H // TGRID) * (W // TGRID)
K = 2048
GC = 128
NCHUNK = N // GC
NWORK = 32
CPW = NCHUNK // NWORK
PB = 512

RCUT = 4.8995


def _bin_body(m2d_ref, op_ref, col_ref, sc_ref, rot_ref, neg_ref,
              p8_ref, cnt_ref, dst_ref, rad_ref):
    theta = rot_ref[0:1, :] * (2.0 * jnp.pi)
    sx = sc_ref[0:1, :] * 0.02 + 1e-3
    sy = sc_ref[1:2, :] * 0.02 + 1e-3
    ct = jnp.cos(theta)
    st = jnp.sin(theta)
    a = ct * ct * sx * sx + st * st * sy * sy
    b = ct * st * (sx * sx - sy * sy)
    c = st * st * sx * sx + ct * ct * sy * sy
    det = a * c - b * b
    A2 = -0.5 * (c / det)
    B2 = b / det
    C2 = -0.5 * (a / det)
    op = jnp.clip(op_ref[0:1, :], 0.0, 0.99) * neg_ref[0:1, :]
    mx = m2d_ref[0:1, :]
    my = m2d_ref[1:2, :]
    smax = jnp.maximum(sx, sy)
    rad_ref[...] = jnp.ceil(3.0 * smax * float(max(H, W))).astype(jnp.int32)
    p8_ref[...] = jnp.concatenate(
        [mx, my, A2, B2, C2,
         op * col_ref[0:1, :], op * col_ref[1:2, :], op * col_ref[2:3, :]],
        axis=0)

    rx = RCUT * jnp.sqrt(a)
    ry = RCUT * jnp.sqrt(c)
    tg = float(TGRID)
    txl = jnp.clip(jnp.floor((mx - rx) * tg).astype(jnp.int32), 0, TGRID - 1)
    txh = jnp.clip(jnp.floor((mx + rx) * tg).astype(jnp.int32), 0, TGRID - 1)
    tyl = jnp.clip(jnp.floor((my - ry) * tg).astype(jnp.int32), 0, TGRID - 1)
    tyh = jnp.clip(jnp.floor((my + ry) * tg).astype(jnp.int32), 0, TGRID - 1)

    t2 = lax.broadcasted_iota(jnp.int32, (T, 1), 0)
    t_x = t2 % TGRID
    t_y = t2 // TGRID
    ov = ((t_x >= txl) & (t_x <= txh) & (t_y >= tyl) & (t_y <= tyh))
    ovf = jnp.where(ov, 1.0, 0.0)

    i1 = lax.broadcasted_iota(jnp.int32, (PB, PB), 0)
    j1 = lax.broadcasted_iota(jnp.int32, (PB, PB), 1)
    lt = jnp.where(i1 < j1, 1.0, 0.0)
    offs = jnp.zeros((T, 1), jnp.float32)
    pieces = []
    for blk in range(N // PB):
        sub = ovf[:, blk * PB:(blk + 1) * PB]
        ppos = lax.dot_general(sub, lt, (((1,), (0,)), ((), ())),
                               precision=lax.Precision.HIGHEST,
                               preferred_element_type=jnp.float32) + offs
        pieces.append(ppos)
        offs = ppos[:, PB - 1:PB] + sub[:, PB - 1:PB]
    pos = jnp.concatenate(pieces, axis=1)
    cnt_ref[...] = jnp.minimum(offs, float(K)).astype(jnp.int32)

    posi = pos.astype(jnp.int32)
    for k in range(9):
        ky, kx = k // 3, k % 3
        t_kx = txl + kx
        t_ky = tyl + ky
        t_k = t_ky * TGRID + t_kx
        onehot = t2 == t_k
        sel = jnp.sum(jnp.where(onehot, posi, 0), axis=0, keepdims=True)
        valid = (t_kx <= txh) & (t_ky <= tyh) & (sel < K)
        gmod = lax.broadcasted_iota(jnp.int32, (1, N), 1) % GC
        trash = T * K + gmod * 9 + k
        dst_ref[k:k + 1, :] = jnp.where(valid, t_k * K + sel, trash)


def _sc_dispatch_body(p_h, dst_h, tp_h, rows_v, dsts_v, sem):
    cid = lax.axis_index("c")
    sid = lax.axis_index("s")
    wid = sid * 2 + cid
    for c in range(CPW):
        chunk = wid * CPW + c
        pltpu.sync_copy(p_h.at[pl.ds(chunk * GC, GC)], rows_v)
        pltpu.sync_copy(dst_h.at[chunk], dsts_v)
        cps = [pltpu.async_copy(rows_v, tp_h.at[dsts_v.at[k]], sem)
               for k in range(9)]
        for cp in cps:
            cp.wait()


RGC = 256


def _raster_body(cnt_sm, tp_ref, img_ref):
    g = pl.program_id(0)
    cnt = cnt_sm[g, 0]
    ty = g // TGRID
    tx = g % TGRID
    i = lax.broadcasted_iota(jnp.int32, (1, TPX), 1)
    tw = W // TGRID
    px = ((tx * tw + (i % tw)).astype(jnp.float32) + 0.5) * (1.0 / W)
    py = ((ty * tw + (i // tw)).astype(jnp.float32) + 0.5) * (1.0 / H)
    rowi = lax.broadcasted_iota(jnp.int32, (RGC, 1), 0)

    def half(base):
        sl = pl.ds(base, RGC)
        mx = tp_ref[0, sl, 0:1]
        my = tp_ref[0, sl, 1:2]
        A2 = tp_ref[0, sl, 2:3]
        B2 = tp_ref[0, sl, 3:4]
        C2 = tp_ref[0, sl, 4:5]
        cT = tp_ref[0, sl, 5:8]
        valid = (base + rowi) < cnt
        cTm = jnp.where(valid, cT, 0.0)
        thr = jnp.where(valid, -12.0, jnp.inf)
        dx = px - mx
        dy = py - my
        power = dx * (A2 * dx + B2 * dy) + C2 * (dy * dy)
        Gv = jnp.where(power > thr, jnp.exp(power), 0.0)
        return lax.dot_general(cTm, Gv, (((0,), (0,)), ((), ())),
                               preferred_element_type=jnp.float32)

    def chunk(j, acc):
        return acc + half(j * 2 * RGC) + half(j * 2 * RGC + RGC)

    nch = (cnt + 2 * RGC - 1) // (2 * RGC)
    acc = lax.fori_loop(0, nch, chunk, jnp.zeros((3, TPX), jnp.float32))
    img_ref[...] = acc.reshape(1, 3, TPX)


@jax.jit
def kernel(means2D, opacities, colors, scale, rots, negative, bg):
    p8, cnts, dst9, rad = pl.pallas_call(
        _bin_body,
        out_shape=[
            jax.ShapeDtypeStruct((8, N), jnp.float32),
            jax.ShapeDtypeStruct((T, 1), jnp.int32),
            jax.ShapeDtypeStruct((9, N), jnp.int32),
            jax.ShapeDtypeStruct((1, N), jnp.int32),
        ],
    )(means2D.T, opacities.T, colors.T, scale.T, rots.T, negative.T)

    ptab = jnp.pad(p8.T, ((0, 0), (0, 128 - 8)))
    dstr = dst9.reshape(9, NCHUNK, GC).transpose(1, 0, 2)

    mesh = plsc.VectorSubcoreMesh(core_axis_name="c", subcore_axis_name="s",
                                  num_cores=2, num_subcores=16)
    tp_flat, = pl.kernel(
        _sc_dispatch_body,
        out_type=[jax.ShapeDtypeStruct(((T + 1) * K, 128), jnp.float32)],
        mesh=mesh,
        scratch_types=[
            pltpu.VMEM((GC, 128), jnp.float32),
            pltpu.VMEM((9, GC), jnp.int32),
            pltpu.SemaphoreType.DMA,
        ],
    )(ptab, dstr)

    tp3 = tp_flat.reshape(T + 1, K, 128)

    img = pl.pallas_call(
        _raster_body,
        grid=(T,),
        in_specs=[
            pl.BlockSpec(memory_space=pltpu.SMEM),
            pl.BlockSpec((1, K, 128), lambda g: (g, 0, 0)),
        ],
        out_specs=pl.BlockSpec((1, 3, TPX), lambda g: (g, 0, 0)),
        out_shape=jax.ShapeDtypeStruct((T, 3, TPX), jnp.float32),
    )(cnts, tp3)

    tw = W // TGRID
    color = (img.reshape(TGRID, TGRID, 3, tw, tw)
             .transpose(2, 0, 3, 1, 4).reshape(3, H, W) + bg[:, None, None])
    return color, rad.reshape(N)


# speedup vs baseline: 11.3306x; 1.0212x over previous
import jax
import jax.numpy as jnp
from jax import lax
from jax.experimental import pallas as pl
from jax.experimental.pallas import tpu as pltpu
from jax.experimental.pallas import tpu_sc as plsc

H = 256
W = 256
N = 8192

TGRID = 8
T = TGRID * TGRID
TPX = (H // TGRID) * (W // TGRID)
K = 2048
GC = 128
NCHUNK = N // GC
NWORK = 32
CPW = NCHUNK // NWORK
PB = 512

RCUT = 4.8995


def _bin_body(m2d_ref, op_ref, col_ref, sc_ref, rot_ref, neg_ref,
              p8_ref, cnt_ref, dst_ref, rad_ref):
    theta = rot_ref[0:1, :] * (2.0 * jnp.pi)
    sx = sc_ref[0:1, :] * 0.02 + 1e-3
    sy = sc_ref[1:2, :] * 0.02 + 1e-3
    ct = jnp.cos(theta)
    st = jnp.sin(theta)
    a = ct * ct * sx * sx + st * st * sy * sy
    b = ct * st * (sx * sx - sy * sy)
    c = st * st * sx * sx + ct * ct * sy * sy
    det = a * c - b * b
    A2 = -0.5 * (c / det)
    B2 = b / det
    C2 = -0.5 * (a / det)
    op = jnp.clip(op_ref[0:1, :], 0.0, 0.99) * neg_ref[0:1, :]
    mx = m2d_ref[0:1, :]
    my = m2d_ref[1:2, :]
    smax = jnp.maximum(sx, sy)
    rad_ref[...] = jnp.ceil(3.0 * smax * float(max(H, W))).astype(jnp.int32)
    p8_ref[...] = jnp.concatenate(
        [mx, my, A2, B2, C2,
         op * col_ref[0:1, :], op * col_ref[1:2, :], op * col_ref[2:3, :]],
        axis=0)

    rx = RCUT * jnp.sqrt(a)
    ry = RCUT * jnp.sqrt(c)
    tg = float(TGRID)
    txl = jnp.clip(jnp.floor((mx - rx) * tg).astype(jnp.int32), 0, TGRID - 1)
    txh = jnp.clip(jnp.floor((mx + rx) * tg).astype(jnp.int32), 0, TGRID - 1)
    tyl = jnp.clip(jnp.floor((my - ry) * tg).astype(jnp.int32), 0, TGRID - 1)
    tyh = jnp.clip(jnp.floor((my + ry) * tg).astype(jnp.int32), 0, TGRID - 1)

    t2 = lax.broadcasted_iota(jnp.int32, (T, 1), 0)
    t_x = t2 % TGRID
    t_y = t2 // TGRID
    ov = ((t_x >= txl) & (t_x <= txh) & (t_y >= tyl) & (t_y <= tyh))
    ovf = jnp.where(ov, 1.0, 0.0)

    i1 = lax.broadcasted_iota(jnp.int32, (PB, PB), 0)
    j1 = lax.broadcasted_iota(jnp.int32, (PB, PB), 1)
    lt = jnp.where(i1 < j1, 1.0, 0.0)
    offs = jnp.zeros((T, 1), jnp.float32)
    pieces = []
    for blk in range(N // PB):
        sub = ovf[:, blk * PB:(blk + 1) * PB]
        ppos = lax.dot_general(sub, lt, (((1,), (0,)), ((), ())),
                               preferred_element_type=jnp.float32) + offs
        pieces.append(ppos)
        offs = ppos[:, PB - 1:PB] + sub[:, PB - 1:PB]
    pos = jnp.concatenate(pieces, axis=1)
    cnt_ref[...] = jnp.minimum(offs, float(K)).astype(jnp.int32)

    posi = pos.astype(jnp.int32)
    for k in range(9):
        ky, kx = k // 3, k % 3
        t_kx = txl + kx
        t_ky = tyl + ky
        t_k = t_ky * TGRID + t_kx
        onehot = t2 == t_k
        sel = jnp.sum(jnp.where(onehot, posi, 0), axis=0, keepdims=True)
        valid = (t_kx <= txh) & (t_ky <= tyh) & (sel < K)
        gmod = lax.broadcasted_iota(jnp.int32, (1, N), 1) % GC
        trash = T * K + gmod * 9 + k
        dst_ref[k:k + 1, :] = jnp.where(valid, t_k * K + sel, trash)


def _sc_dispatch_body(p_h, dst_h, tp_h, rows_v, dsts_v, sem):
    cid = lax.axis_index("c")
    sid = lax.axis_index("s")
    wid = sid * 2 + cid
    for c in range(CPW):
        chunk = wid * CPW + c
        pltpu.sync_copy(p_h.at[pl.ds(chunk * GC, GC)], rows_v)
        pltpu.sync_copy(dst_h.at[chunk], dsts_v)
        cps = [pltpu.async_copy(rows_v, tp_h.at[dsts_v.at[k]], sem)
               for k in range(9)]
        for cp in cps:
            cp.wait()


RGC = 256


def _raster_body(cnt_sm, tp_ref, img_ref):
    g = pl.program_id(0)
    cnt = cnt_sm[g, 0]
    ty = g // TGRID
    tx = g % TGRID
    i = lax.broadcasted_iota(jnp.int32, (1, TPX), 1)
    tw = W // TGRID
    px = ((tx * tw + (i % tw)).astype(jnp.float32) + 0.5) * (1.0 / W)
    py = ((ty * tw + (i // tw)).astype(jnp.float32) + 0.5) * (1.0 / H)
    rowi = lax.broadcasted_iota(jnp.int32, (RGC, 1), 0)

    def half(base):
        sl = pl.ds(base, RGC)
        mx = tp_ref[0, sl, 0:1]
        my = tp_ref[0, sl, 1:2]
        A2 = tp_ref[0, sl, 2:3]
        B2 = tp_ref[0, sl, 3:4]
        C2 = tp_ref[0, sl, 4:5]
        cT = tp_ref[0, sl, 5:8]
        valid = (base + rowi) < cnt
        cTm = jnp.where(valid, cT, 0.0)
        thr = jnp.where(valid, -12.0, jnp.inf)
        dx = px - mx
        dy = py - my
        power = dx * (A2 * dx + B2 * dy) + C2 * (dy * dy)
        Gv = jnp.where(power > thr, jnp.exp(power), 0.0)
        return lax.dot_general(cTm, Gv, (((0,), (0,)), ((), ())),
                               preferred_element_type=jnp.float32)

    def chunk(j, acc):
        return acc + half(j * 2 * RGC) + half(j * 2 * RGC + RGC)

    nch = (cnt + 2 * RGC - 1) // (2 * RGC)
    acc = lax.fori_loop(0, nch, chunk, jnp.zeros((3, TPX), jnp.float32))
    img_ref[...] = acc.reshape(1, 3, TPX)


@jax.jit
def kernel(means2D, opacities, colors, scale, rots, negative, bg):
    p8, cnts, dst9, rad = pl.pallas_call(
        _bin_body,
        out_shape=[
            jax.ShapeDtypeStruct((8, N), jnp.float32),
            jax.ShapeDtypeStruct((T, 1), jnp.int32),
            jax.ShapeDtypeStruct((9, N), jnp.int32),
            jax.ShapeDtypeStruct((1, N), jnp.int32),
        ],
    )(means2D.T, opacities.T, colors.T, scale.T, rots.T, negative.T)

    ptab = jnp.pad(p8.T, ((0, 0), (0, 128 - 8)))
    dstr = dst9.reshape(9, NCHUNK, GC).transpose(1, 0, 2)

    mesh = plsc.VectorSubcoreMesh(core_axis_name="c", subcore_axis_name="s",
                                  num_cores=2, num_subcores=16)
    tp_flat, = pl.kernel(
        _sc_dispatch_body,
        out_type=[jax.ShapeDtypeStruct(((T + 1) * K, 128), jnp.float32)],
        mesh=mesh,
        scratch_types=[
            pltpu.VMEM((GC, 128), jnp.float32),
            pltpu.VMEM((9, GC), jnp.int32),
            pltpu.SemaphoreType.DMA,
        ],
    )(ptab, dstr)

    tp3 = tp_flat.reshape(T + 1, K, 128)

    img = pl.pallas_call(
        _raster_body,
        grid=(T,),
        in_specs=[
            pl.BlockSpec(memory_space=pltpu.SMEM),
            pl.BlockSpec((1, K, 128), lambda g: (g, 0, 0)),
        ],
        out_specs=pl.BlockSpec((1, 3, TPX), lambda g: (g, 0, 0)),
        out_shape=jax.ShapeDtypeStruct((T, 3, TPX), jnp.float32),
    )(cnts, tp3)

    tw = W // TGRID
    color = (img.reshape(TGRID, TGRID, 3, tw, tw)
             .transpose(2, 0, 3, 1, 4).reshape(3, H, W) + bg[:, None, None])
    return color, rad.reshape(N)
